# baseline (speedup 1.0000x reference)
import functools
import math

import numpy as np

import jax
import jax.numpy as jnp
from jax import lax
from jax.experimental import pallas as pl
from jax.experimental.pallas import tpu as pltpu
from jax.experimental.pallas import tpu_sc as plsc

_LANES = 16
_CHUNK = 100


def _positional_encoding(length: int, dim: int) -> np.ndarray:
    position = np.arange(0, length, dtype=np.float64)[:, None]
    div_term = np.exp(
        np.arange(0.0, dim, 2, dtype=np.float64) * -(math.log(10000.0) / dim)
    )
    tmp = position * div_term
    pe = np.zeros((length, dim), dtype=np.float64)
    pe[:, 0::2] = np.sin(tmp)
    pe[:, 1::2] = np.cos(tmp)
    return pe.astype(np.float32)


def _embed_body(num_cores, rows_per_w, n_chunks, dim, scale,
                idx_hbm, pe_hbm, table_hbm, out_hbm,
                idx_v, pe_v, *rest):
    nbuf = 8
    bufs = rest[0:nbuf]
    gsems = rest[nbuf:2 * nbuf]
    osems = rest[2 * nbuf:3 * nbuf]

    w = lax.axis_index("s") * num_cores + lax.axis_index("c")
    base_row = w * rows_per_w

    pltpu.sync_copy(idx_hbm.at[w], idx_v)
    pltpu.sync_copy(pe_hbm, pe_v)

    def start_gather(c, slot):
        pltpu.async_copy(table_hbm.at[idx_v.at[c]], bufs[slot], gsems[slot])

    def out_dst(c):
        return out_hbm.at[pl.ds(base_row + c * _CHUNK, _CHUNK), pl.ds(0, dim)]

    for c0 in range(nbuf - 1):
        start_gather(c0, c0)

    def outer(g, carry):
        for j in range(nbuf):
            c = g * nbuf + j
            buf, gsem, osem = bufs[j], gsems[j], osems[j]
            pltpu.make_async_copy(table_hbm.at[idx_v.at[c]], buf, gsem).wait()
            pbase = (j % 2) * _CHUNK

            def inner(r, acc):
                row = pbase + r
                for k in range(dim // _LANES):
                    sl = pl.ds(k * _LANES, _LANES)
                    buf[r, sl] = buf[r, sl] * scale + pe_v[row, sl]
                return acc

            lax.fori_loop(0, _CHUNK, inner, 0, unroll=4)

            pltpu.async_copy(buf, out_dst(c), osem)

            nxt = c + nbuf - 1
            nslot = (j + nbuf - 1) % nbuf

            @pl.when(nxt < n_chunks)
            def _():
                @pl.when(nxt >= nbuf)
                def _():
                    pltpu.make_async_copy(
                        bufs[nslot], out_dst(0), osems[nslot]
                    ).wait()

                start_gather(nxt, nslot)

        return carry

    lax.fori_loop(0, n_chunks // nbuf, outer, 0)

    for slot in range(nbuf):
        pltpu.make_async_copy(bufs[slot], out_dst(0), osems[slot]).wait()


def kernel(input, table):
    batch, seq = input.shape
    vocab, dim = table.shape
    scale = float(math.sqrt(dim))

    info = plsc.get_sparse_core_info()
    nw = info.num_cores * info.num_subcores

    assert seq == 2 * _CHUNK and batch % nw == 0 and dim == 64
    rows = batch * seq
    rows_per_w = rows // nw
    n_chunks = rows_per_w // _CHUNK
    assert n_chunks % 8 == 0

    pe = jnp.asarray(_positional_encoding(seq, dim))
    table2 = jnp.pad(table, ((0, 0), (0, 128 - dim))).reshape(2 * vocab, dim)
    idx = (input.astype(jnp.int32) * 2).reshape(nw, n_chunks, _CHUNK)

    mesh = plsc.VectorSubcoreMesh(core_axis_name="c", subcore_axis_name="s")
    body = functools.partial(
        _embed_body, info.num_cores, rows_per_w, n_chunks, dim, scale
    )
    call = pl.kernel(
        body,
        out_type=jax.ShapeDtypeStruct((rows, 2 * dim), jnp.float32),
        mesh=mesh,
        scratch_types=(
            [pltpu.VMEM((n_chunks, _CHUNK), jnp.int32),
             pltpu.VMEM((seq, dim), jnp.float32)]
            + [pltpu.VMEM((_CHUNK, dim), jnp.float32)] * 8
            + [pltpu.SemaphoreType.DMA] * 16
        ),
        compiler_params=pltpu.CompilerParams(use_tc_tiling_on_sc=False),
        name="sc_embedding_lookup",
    )
    out = call(idx, pe, table2)
    return out[:, :dim].reshape(batch, seq, dim)


# speedup vs baseline: 1.3658x; 1.3658x over previous
import functools
import math

import numpy as np

import jax
import jax.numpy as jnp
from jax import lax
from jax.experimental import pallas as pl
from jax.experimental.pallas import tpu as pltpu
from jax.experimental.pallas import tpu_sc as plsc

_LANES = 16
_CHUNK = 100


def _positional_encoding(length: int, dim: int) -> np.ndarray:
    position = np.arange(0, length, dtype=np.float64)[:, None]
    div_term = np.exp(
        np.arange(0.0, dim, 2, dtype=np.float64) * -(math.log(10000.0) / dim)
    )
    tmp = position * div_term
    pe = np.zeros((length, dim), dtype=np.float64)
    pe[:, 0::2] = np.sin(tmp)
    pe[:, 1::2] = np.cos(tmp)
    return pe.astype(np.float32)


def _embed_body(num_cores, rows_per_w, n_chunks, dim, scale,
                idx_hbm, pe_hbm, table_hbm, out_hbm,
                idx_v, pe_v, *rest):
    nbuf = 8
    bufs = rest[0:nbuf]
    gsems = rest[nbuf:2 * nbuf]
    osems = rest[2 * nbuf:3 * nbuf]

    w = lax.axis_index("s") * num_cores + lax.axis_index("c")
    base_row = w * rows_per_w

    pltpu.sync_copy(idx_hbm.at[w], idx_v)
    pltpu.sync_copy(pe_hbm, pe_v)

    def start_gather(c, slot):
        pltpu.async_copy(table_hbm.at[idx_v.at[c]], bufs[slot], gsems[slot])

    def out_dst(c):
        return out_hbm.at[pl.ds(base_row + c * _CHUNK, _CHUNK), pl.ds(0, dim)]

    for c0 in range(nbuf - 1):
        start_gather(c0, c0)

    def outer(g, carry):
        for j in range(nbuf):
            c = g * nbuf + j
            buf, gsem, osem = bufs[j], gsems[j], osems[j]
            pltpu.make_async_copy(table_hbm.at[idx_v.at[c]], buf, gsem).wait()
            pbase = (j % 2) * _CHUNK

            def inner(r, acc):
                row = pbase + r
                for k in range(dim // _LANES):
                    sl = pl.ds(k * _LANES, _LANES)
                    buf[r, sl] = buf[r, sl] * scale + pe_v[row, sl]
                return acc

            lax.fori_loop(0, _CHUNK, inner, 0)

            pltpu.async_copy(buf, out_dst(c), osem)

            nxt = c + nbuf - 1
            nslot = (j + nbuf - 1) % nbuf

            @pl.when(nxt < n_chunks)
            def _():
                @pl.when(nxt >= nbuf)
                def _():
                    pltpu.make_async_copy(
                        bufs[nslot], out_dst(0), osems[nslot]
                    ).wait()

                start_gather(nxt, nslot)

        return carry

    lax.fori_loop(0, n_chunks // nbuf, outer, 0)

    for slot in range(nbuf):
        pltpu.make_async_copy(bufs[slot], out_dst(0), osems[slot]).wait()


def kernel(input, table):
    batch, seq = input.shape
    vocab, dim = table.shape
    scale = float(math.sqrt(dim))

    info = plsc.get_sparse_core_info()
    nw = info.num_cores * info.num_subcores

    assert seq == 2 * _CHUNK and batch % nw == 0 and dim == 64
    rows = batch * seq
    rows_per_w = rows // nw
    n_chunks = rows_per_w // _CHUNK
    assert n_chunks % 8 == 0

    pe = jnp.asarray(_positional_encoding(seq, dim))
    table2 = jnp.pad(table, ((0, 0), (0, 128 - dim))).reshape(2 * vocab, dim)
    idx = (input.astype(jnp.int32) * 2).reshape(nw, n_chunks, _CHUNK)

    mesh = plsc.VectorSubcoreMesh(core_axis_name="c", subcore_axis_name="s")
    body = functools.partial(
        _embed_body, info.num_cores, rows_per_w, n_chunks, dim, scale
    )
    call = pl.kernel(
        body,
        out_type=jax.ShapeDtypeStruct((rows, 2 * dim), jnp.float32),
        mesh=mesh,
        scratch_types=(
            [pltpu.VMEM((n_chunks, _CHUNK), jnp.int32),
             pltpu.VMEM((seq, dim), jnp.float32)]
            + [pltpu.VMEM((_CHUNK, dim), jnp.float32)] * 8
            + [pltpu.SemaphoreType.DMA] * 16
        ),
        compiler_params=pltpu.CompilerParams(use_tc_tiling_on_sc=False),
        name="sc_embedding_lookup",
    )
    out = call(idx, pe, table2)
    return out[:, :dim].reshape(batch, seq, dim)


# speedup vs baseline: 1.7162x; 1.2565x over previous
import functools
import math

import numpy as np

import jax
import jax.numpy as jnp
from jax import lax
from jax.experimental import pallas as pl
from jax.experimental.pallas import tpu as pltpu
from jax.experimental.pallas import tpu_sc as plsc

_LANES = 16
_CHUNK = 100


def _positional_encoding(length: int, dim: int) -> np.ndarray:
    position = np.arange(0, length, dtype=np.float64)[:, None]
    div_term = np.exp(
        np.arange(0.0, dim, 2, dtype=np.float64) * -(math.log(10000.0) / dim)
    )
    tmp = position * div_term
    pe = np.zeros((length, dim), dtype=np.float64)
    pe[:, 0::2] = np.sin(tmp)
    pe[:, 1::2] = np.cos(tmp)
    return pe.astype(np.float32)


def _embed_body(num_cores, rows_per_w, n_chunks, dim, scale,
                idx_hbm, pe_hbm, table_hbm, out_hbm,
                idx_v, pe_v, *rest):
    nbuf = 8
    bufs = rest[0:nbuf]
    gsems = rest[nbuf:2 * nbuf]
    osems = rest[2 * nbuf:3 * nbuf]

    w = lax.axis_index("s") * num_cores + lax.axis_index("c")
    base_row = w * rows_per_w

    pltpu.sync_copy(idx_hbm.at[w], idx_v)
    pltpu.sync_copy(pe_hbm, pe_v)

    def start_gather(c, slot):
        pltpu.async_copy(table_hbm.at[idx_v.at[c]], bufs[slot], gsems[slot])

    def out_dst(c):
        return out_hbm.at[pl.ds(base_row + c * _CHUNK, _CHUNK), pl.ds(0, dim)]

    for c0 in range(nbuf - 1):
        start_gather(c0, c0)

    def outer(g, carry):
        for j in range(nbuf):
            c = g * nbuf + j
            buf, gsem, osem = bufs[j], gsems[j], osems[j]
            pltpu.make_async_copy(table_hbm.at[idx_v.at[c]], buf, gsem).wait()
            pbase = (j % 2) * _CHUNK

            def inner(r, acc):
                row = pbase + r
                for k in range(dim // _LANES):
                    sl = pl.ds(k * _LANES, _LANES)
                    buf[r, sl] = buf[r, sl] * scale + pe_v[row, sl]
                return acc

            lax.fori_loop(0, _CHUNK, inner, 0)

            pltpu.async_copy(buf, out_dst(c), osem)

            nxt = c + nbuf - 1
            nslot = (j + nbuf - 1) % nbuf

            @pl.when(nxt < n_chunks)
            def _():
                @pl.when(nxt >= nbuf)
                def _():
                    pltpu.make_async_copy(
                        bufs[nslot], out_dst(0), osems[nslot]
                    ).wait()

                start_gather(nxt, nslot)

        return carry

    lax.fori_loop(0, n_chunks // nbuf, outer, 0)

    for slot in range(nbuf):
        pltpu.make_async_copy(bufs[slot], out_dst(0), osems[slot]).wait()


def _transpose_pad_table(table, vocab, dim):
    br = 4096

    def body(tt_ref, out_ref):
        xt = tt_ref[...].T
        out_ref[...] = jnp.concatenate(
            [xt, jnp.zeros((br, 128 - dim), jnp.float32)], axis=1
        )

    return pl.pallas_call(
        body,
        grid=(pl.cdiv(vocab, br),),
        in_specs=[pl.BlockSpec((dim, br), lambda i: (0, i))],
        out_specs=pl.BlockSpec((br, 128), lambda i: (i, 0)),
        out_shape=jax.ShapeDtypeStruct((vocab, 128), jnp.float32),
    )(table.T)


def kernel(input, table):
    batch, seq = input.shape
    vocab, dim = table.shape
    scale = float(math.sqrt(dim))

    info = plsc.get_sparse_core_info()
    nw = info.num_cores * info.num_subcores

    assert seq == 2 * _CHUNK and batch % nw == 0 and dim == 64
    rows = batch * seq
    rows_per_w = rows // nw
    n_chunks = rows_per_w // _CHUNK
    assert n_chunks % 8 == 0

    pe = jnp.asarray(_positional_encoding(seq, dim))
    table2 = _transpose_pad_table(table, vocab, dim).reshape(2 * vocab, dim)
    idx = (input.astype(jnp.int32) * 2).reshape(nw, n_chunks, _CHUNK)

    mesh = plsc.VectorSubcoreMesh(core_axis_name="c", subcore_axis_name="s")
    body = functools.partial(
        _embed_body, info.num_cores, rows_per_w, n_chunks, dim, scale
    )
    call = pl.kernel(
        body,
        out_type=jax.ShapeDtypeStruct((rows, 2 * dim), jnp.float32),
        mesh=mesh,
        scratch_types=(
            [pltpu.VMEM((n_chunks, _CHUNK), jnp.int32),
             pltpu.VMEM((seq, dim), jnp.float32)]
            + [pltpu.VMEM((_CHUNK, dim), jnp.float32)] * 8
            + [pltpu.SemaphoreType.DMA] * 16
        ),
        compiler_params=pltpu.CompilerParams(use_tc_tiling_on_sc=False),
        name="sc_embedding_lookup",
    )
    out = call(idx, pe, table2)
    return out[:, :dim].reshape(batch, seq, dim)


# speedup vs baseline: 1.9135x; 1.1150x over previous
import functools
import math

import numpy as np

import jax
import jax.numpy as jnp
from jax import lax
from jax.experimental import pallas as pl
from jax.experimental.pallas import tpu as pltpu
from jax.experimental.pallas import tpu_sc as plsc

_LANES = 16
_CHUNK = 100


def _positional_encoding(length: int, dim: int) -> np.ndarray:
    position = np.arange(0, length, dtype=np.float64)[:, None]
    div_term = np.exp(
        np.arange(0.0, dim, 2, dtype=np.float64) * -(math.log(10000.0) / dim)
    )
    tmp = position * div_term
    pe = np.zeros((length, dim), dtype=np.float64)
    pe[:, 0::2] = np.sin(tmp)
    pe[:, 1::2] = np.cos(tmp)
    return pe.astype(np.float32)


def _embed_body(num_cores, rows_per_w, n_chunks, dim, scale,
                idx_hbm, pe_hbm, table_hbm, out_hbm,
                idx_v, pe_v, *rest):
    nbuf = 8
    bufs = rest[0:nbuf]
    gsems = rest[nbuf:2 * nbuf]
    osems = rest[2 * nbuf:3 * nbuf]

    w = lax.axis_index("s") * num_cores + lax.axis_index("c")
    base_row = w * rows_per_w

    pltpu.sync_copy(idx_hbm.at[w], idx_v)
    pltpu.sync_copy(pe_hbm, pe_v)

    def start_gather(c, slot):
        pltpu.async_copy(table_hbm.at[idx_v.at[c]], bufs[slot], gsems[slot])

    def out_dst(c):
        return out_hbm.at[pl.ds(base_row + c * _CHUNK, _CHUNK), pl.ds(0, dim)]

    for c0 in range(nbuf - 1):
        start_gather(c0, c0)

    def outer(g, carry):
        for j in range(nbuf):
            c = g * nbuf + j
            buf, gsem, osem = bufs[j], gsems[j], osems[j]
            pltpu.make_async_copy(table_hbm.at[idx_v.at[c]], buf, gsem).wait()
            pbase = (j % 2) * _CHUNK

            def inner(r, acc):
                row = pbase + r
                for k in range(dim // _LANES):
                    sl = pl.ds(k * _LANES, _LANES)
                    buf[r, sl] = buf[r, sl] * scale + pe_v[row, sl]
                return acc

            lax.fori_loop(0, _CHUNK, inner, 0)

            pltpu.async_copy(buf, out_dst(c), osem)

            nxt = c + nbuf - 1
            nslot = (j + nbuf - 1) % nbuf

            @pl.when(nxt < n_chunks)
            def _():
                @pl.when(nxt >= nbuf)
                def _():
                    pltpu.make_async_copy(
                        bufs[nslot], out_dst(0), osems[nslot]
                    ).wait()

                start_gather(nxt, nslot)

        return carry

    lax.fori_loop(0, n_chunks // nbuf, outer, 0)

    for slot in range(nbuf):
        pltpu.make_async_copy(bufs[slot], out_dst(0), osems[slot]).wait()


def _transpose_pad_table(table, vocab, dim):
    br = 8192

    def body(tt_ref, out_ref):
        out_ref[:, :dim] = tt_ref[...].T

    return pl.pallas_call(
        body,
        grid=(pl.cdiv(vocab, br),),
        in_specs=[pl.BlockSpec((dim, br), lambda i: (0, i))],
        out_specs=pl.BlockSpec((br, 128), lambda i: (i, 0)),
        out_shape=jax.ShapeDtypeStruct((vocab, 128), jnp.float32),
    )(table.T)


def kernel(input, table):
    batch, seq = input.shape
    vocab, dim = table.shape
    scale = float(math.sqrt(dim))

    info = plsc.get_sparse_core_info()
    nw = info.num_cores * info.num_subcores

    assert seq == 2 * _CHUNK and batch % nw == 0 and dim == 64
    rows = batch * seq
    rows_per_w = rows // nw
    n_chunks = rows_per_w // _CHUNK
    assert n_chunks % 8 == 0

    pe = jnp.asarray(_positional_encoding(seq, dim))
    table2 = _transpose_pad_table(table, vocab, dim).reshape(2 * vocab, dim)
    idx = (input.astype(jnp.int32) * 2).reshape(nw, n_chunks, _CHUNK)

    mesh = plsc.VectorSubcoreMesh(core_axis_name="c", subcore_axis_name="s")
    body = functools.partial(
        _embed_body, info.num_cores, rows_per_w, n_chunks, dim, scale
    )
    call = pl.kernel(
        body,
        out_type=jax.ShapeDtypeStruct((rows, 2 * dim), jnp.float32),
        mesh=mesh,
        scratch_types=(
            [pltpu.VMEM((n_chunks, _CHUNK), jnp.int32),
             pltpu.VMEM((seq, dim), jnp.float32)]
            + [pltpu.VMEM((_CHUNK, dim), jnp.float32)] * 8
            + [pltpu.SemaphoreType.DMA] * 16
        ),
        compiler_params=pltpu.CompilerParams(use_tc_tiling_on_sc=False),
        name="sc_embedding_lookup",
    )
    out = call(idx, pe, table2)
    return out[:, :dim].reshape(batch, seq, dim)


# speedup vs baseline: 1.9736x; 1.0314x over previous
import functools
import math

import numpy as np

import jax
import jax.numpy as jnp
from jax import lax
from jax.experimental import pallas as pl
from jax.experimental.pallas import tpu as pltpu
from jax.experimental.pallas import tpu_sc as plsc

_LANES = 16
_CHUNK = 100


def _positional_encoding(length: int, dim: int) -> np.ndarray:
    position = np.arange(0, length, dtype=np.float64)[:, None]
    div_term = np.exp(
        np.arange(0.0, dim, 2, dtype=np.float64) * -(math.log(10000.0) / dim)
    )
    tmp = position * div_term
    pe = np.zeros((length, dim), dtype=np.float64)
    pe[:, 0::2] = np.sin(tmp)
    pe[:, 1::2] = np.cos(tmp)
    return pe.astype(np.float32)


def _embed_body(num_cores, rows_per_w, n_chunks, dim, scale,
                idx_hbm, pe_hbm, table_hbm, out_hbm,
                idx_v, pe_v, *rest):
    nbuf = 8
    bufs = rest[0:nbuf]
    gsems = rest[nbuf:2 * nbuf]
    osems = rest[2 * nbuf:3 * nbuf]

    w = lax.axis_index("s") * num_cores + lax.axis_index("c")
    base_row = w * rows_per_w

    pltpu.sync_copy(idx_hbm.at[w], idx_v)
    pltpu.sync_copy(pe_hbm, pe_v)

    def start_gather(c, slot):
        pltpu.async_copy(table_hbm.at[idx_v.at[c]], bufs[slot], gsems[slot])

    def out_dst(c):
        return out_hbm.at[pl.ds(base_row + c * _CHUNK, _CHUNK), pl.ds(0, dim)]

    for c0 in range(nbuf - 1):
        start_gather(c0, c0)

    def outer(g, carry):
        for j in range(nbuf):
            c = g * nbuf + j
            buf, gsem, osem = bufs[j], gsems[j], osems[j]
            pltpu.make_async_copy(table_hbm.at[idx_v.at[c]], buf, gsem).wait()
            pbase = (j % 2) * _CHUNK

            def inner(r, acc):
                row = pbase + r
                for k in range(dim // _LANES):
                    sl = pl.ds(k * _LANES, _LANES)
                    buf[r, sl] = buf[r, sl] * scale + pe_v[row, sl]
                return acc

            lax.fori_loop(0, _CHUNK, inner, 0)

            pltpu.async_copy(buf, out_dst(c), osem)

            nxt = c + nbuf - 1
            nslot = (j + nbuf - 1) % nbuf

            @pl.when(nxt < n_chunks)
            def _():
                @pl.when(nxt >= nbuf)
                def _():
                    pltpu.make_async_copy(
                        bufs[nslot], out_dst(0), osems[nslot]
                    ).wait()

                start_gather(nxt, nslot)

        return carry

    lax.fori_loop(0, n_chunks // nbuf, outer, 0)

    for slot in range(nbuf):
        pltpu.make_async_copy(bufs[slot], out_dst(0), osems[slot]).wait()


def _transpose_pad_table(table, vocab, dim):
    br = 16384

    def body(tt_ref, out_ref):
        out_ref[:, :dim] = tt_ref[...].T

    return pl.pallas_call(
        body,
        grid=(pl.cdiv(vocab, br),),
        in_specs=[pl.BlockSpec((dim, br), lambda i: (0, i))],
        out_specs=pl.BlockSpec((br, 128), lambda i: (i, 0)),
        out_shape=jax.ShapeDtypeStruct((vocab, 128), jnp.float32),
    )(table.T)


def kernel(input, table):
    batch, seq = input.shape
    vocab, dim = table.shape
    scale = float(math.sqrt(dim))

    info = plsc.get_sparse_core_info()
    nw = info.num_cores * info.num_subcores

    assert seq == 2 * _CHUNK and batch % nw == 0 and dim == 64
    rows = batch * seq
    rows_per_w = rows // nw
    n_chunks = rows_per_w // _CHUNK
    assert n_chunks % 8 == 0

    pe = jnp.asarray(_positional_encoding(seq, dim))
    table2 = _transpose_pad_table(table, vocab, dim).reshape(2 * vocab, dim)
    idx = (input.astype(jnp.int32) * 2).reshape(nw, n_chunks, _CHUNK)

    mesh = plsc.VectorSubcoreMesh(core_axis_name="c", subcore_axis_name="s")
    body = functools.partial(
        _embed_body, info.num_cores, rows_per_w, n_chunks, dim, scale
    )
    call = pl.kernel(
        body,
        out_type=jax.ShapeDtypeStruct((rows, 2 * dim), jnp.float32),
        mesh=mesh,
        scratch_types=(
            [pltpu.VMEM((n_chunks, _CHUNK), jnp.int32),
             pltpu.VMEM((seq, dim), jnp.float32)]
            + [pltpu.VMEM((_CHUNK, dim), jnp.float32)] * 8
            + [pltpu.SemaphoreType.DMA] * 16
        ),
        compiler_params=pltpu.CompilerParams(use_tc_tiling_on_sc=False),
        name="sc_embedding_lookup",
    )
    out = call(idx, pe, table2)
    return out[:, :dim].reshape(batch, seq, dim)


# speedup vs baseline: 1.9978x; 1.0122x over previous
import functools
import math

import numpy as np

import jax
import jax.numpy as jnp
from jax import lax
from jax.experimental import pallas as pl
from jax.experimental.pallas import tpu as pltpu
from jax.experimental.pallas import tpu_sc as plsc

_LANES = 16
_CHUNK = 100


def _positional_encoding(length: int, dim: int) -> np.ndarray:
    position = np.arange(0, length, dtype=np.float64)[:, None]
    div_term = np.exp(
        np.arange(0.0, dim, 2, dtype=np.float64) * -(math.log(10000.0) / dim)
    )
    tmp = position * div_term
    pe = np.zeros((length, dim), dtype=np.float64)
    pe[:, 0::2] = np.sin(tmp)
    pe[:, 1::2] = np.cos(tmp)
    return pe.astype(np.float32)


def _embed_body(num_cores, rows_per_w, n_chunks, dim, scale,
                idx_hbm, pe_hbm, table_hbm, out_hbm,
                idx_v, pe_v, *rest):
    nbuf = 8
    bufs = rest[0:nbuf]
    gsems = rest[nbuf:2 * nbuf]
    osems = rest[2 * nbuf:3 * nbuf]

    w = lax.axis_index("s") * num_cores + lax.axis_index("c")
    base_row = w * rows_per_w

    pltpu.sync_copy(idx_hbm.at[w], idx_v)
    pltpu.sync_copy(pe_hbm, pe_v)

    def start_gather(c, slot):
        pltpu.async_copy(table_hbm.at[idx_v.at[c]], bufs[slot], gsems[slot])

    def out_dst(c):
        return out_hbm.at[pl.ds(base_row + c * _CHUNK, _CHUNK), pl.ds(0, dim)]

    for c0 in range(nbuf - 1):
        start_gather(c0, c0)

    def outer(g, carry):
        for j in range(nbuf):
            c = g * nbuf + j
            buf, gsem, osem = bufs[j], gsems[j], osems[j]
            pltpu.make_async_copy(table_hbm.at[idx_v.at[c]], buf, gsem).wait()
            pbase = (j % 2) * _CHUNK

            def inner(r, acc):
                row = pbase + r
                for k in range(dim // _LANES):
                    sl = pl.ds(k * _LANES, _LANES)
                    buf[r, sl] = buf[r, sl] * scale + pe_v[row, sl]
                return acc

            lax.fori_loop(0, _CHUNK, inner, 0)

            pltpu.async_copy(buf, out_dst(c), osem)

            nxt = c + nbuf - 1
            nslot = (j + nbuf - 1) % nbuf

            @pl.when(nxt < n_chunks)
            def _():
                @pl.when(nxt >= nbuf)
                def _():
                    pltpu.make_async_copy(
                        bufs[nslot], out_dst(0), osems[nslot]
                    ).wait()

                start_gather(nxt, nslot)

        return carry

    lax.fori_loop(0, n_chunks // nbuf, outer, 0)

    for slot in range(nbuf):
        pltpu.make_async_copy(bufs[slot], out_dst(0), osems[slot]).wait()


def _transpose_pad_table(table, vocab, dim):
    br = 32768

    def body(tt_ref, out_ref):
        out_ref[:, :dim] = tt_ref[...].T

    return pl.pallas_call(
        body,
        grid=(pl.cdiv(vocab, br),),
        in_specs=[pl.BlockSpec((dim, br), lambda i: (0, i))],
        out_specs=pl.BlockSpec((br, 128), lambda i: (i, 0)),
        out_shape=jax.ShapeDtypeStruct((vocab, 128), jnp.float32),
    )(table.T)


def kernel(input, table):
    batch, seq = input.shape
    vocab, dim = table.shape
    scale = float(math.sqrt(dim))

    info = plsc.get_sparse_core_info()
    nw = info.num_cores * info.num_subcores

    assert seq == 2 * _CHUNK and batch % nw == 0 and dim == 64
    rows = batch * seq
    rows_per_w = rows // nw
    n_chunks = rows_per_w // _CHUNK
    assert n_chunks % 8 == 0

    pe = jnp.asarray(_positional_encoding(seq, dim))
    table2 = _transpose_pad_table(table, vocab, dim).reshape(2 * vocab, dim)
    idx = (input.astype(jnp.int32) * 2).reshape(nw, n_chunks, _CHUNK)

    mesh = plsc.VectorSubcoreMesh(core_axis_name="c", subcore_axis_name="s")
    body = functools.partial(
        _embed_body, info.num_cores, rows_per_w, n_chunks, dim, scale
    )
    call = pl.kernel(
        body,
        out_type=jax.ShapeDtypeStruct((rows, 2 * dim), jnp.float32),
        mesh=mesh,
        scratch_types=(
            [pltpu.VMEM((n_chunks, _CHUNK), jnp.int32),
             pltpu.VMEM((seq, dim), jnp.float32)]
            + [pltpu.VMEM((_CHUNK, dim), jnp.float32)] * 8
            + [pltpu.SemaphoreType.DMA] * 16
        ),
        compiler_params=pltpu.CompilerParams(use_tc_tiling_on_sc=False),
        name="sc_embedding_lookup",
    )
    out = call(idx, pe, table2)
    return out[:, :dim].reshape(batch, seq, dim)
